# baseline (speedup 1.0000x reference)
import numpy as np

import jax
import jax.numpy as jnp
from jax.experimental import pallas as pl
from jax.experimental.pallas import tpu as pltpu
from jax.sharding import Mesh, PartitionSpec as P

W = 8
CHUNK = 8
NS_EPS = 1e-7


def _dot(a, b, dims):
    return jax.lax.dot_general(a, b, (dims, ((), ())),
                               preferred_element_type=jnp.float32)


def _atlas_kernel(x_ref, k_ref, v_ref, Mp_ref, Sp_ref, pc_ref,
                  aW_ref, ab_ref, eW_ref, eb_ref, tW_ref, tb_ref,
                  gW_ref, gb_ref, oW_ref, ob_ref,
                  out_ref, M_ref, S_ref,
                  bk_s, bv_s):
    pair = x_ref.shape[0]
    c = pl.program_id(0)

    @pl.when(c == 0)
    def _init():
        M_ref[...] = Mp_ref[...]
        S_ref[...] = Sp_ref[...]
        bk_s[...] = jnp.zeros_like(bk_s)
        bv_s[...] = jnp.zeros_like(bv_s)

    kphi_p, ks_p, vs_p, aT_p, eT_p, tT_p = [], [], [], [], [], []
    for i in range(pair):
        x_c = x_ref[i]
        k_c = k_ref[i]
        v_c = v_ref[i]
        kphi = pc_ref[0:1, :] * k_c + pc_ref[1:2, :] * (k_c * k_c)
        aT_p.append(jax.nn.sigmoid(_dot(aW_ref[...], x_c, ((1,), (1,))) + ab_ref[...]))
        eT_p.append(jax.nn.sigmoid(_dot(eW_ref[...], x_c, ((1,), (1,))) + eb_ref[...]) * 0.1)
        tT_p.append(jax.nn.sigmoid(_dot(tW_ref[...], x_c, ((1,), (1,))) + tb_ref[...]))
        g = jax.nn.sigmoid(_dot(x_c, gW_ref[...], ((1,), (0,))) + gb_ref[...])
        sg = jnp.sqrt(g)
        kphi_p.append(kphi)
        ks_p.append(sg * kphi)
        vs_p.append(sg * v_c)

    M_p = [M_ref[i] for i in range(pair)]
    S_p = [S_ref[i] for i in range(pair)]
    ys_p = [[] for _ in range(pair)]
    for j in range(CHUNK):
        rc = jnp.where(c == 0, jnp.float32(1.0 / (j + 1)), jnp.float32(1.0 / W))
        for i in range(pair):
            bk_s[i, j:j + 1, :] = ks_p[i][j:j + 1, :]
            bv_s[i, j:j + 1, :] = vs_p[i][j:j + 1, :]
            Kw = bk_s[i]
            Vw = bv_s[i]
            M, S = M_p[i], S_p[i]
            pe = (_dot(Kw, M, ((1,), (1,))) - Vw) * rc
            grad = _dot(pe, Kw, ((0,), (0,)))
            S = tT_p[i][:, j:j + 1] * S + grad
            nrm = jnp.sqrt(jnp.sum(S * S)) + NS_EPS
            SSt = _dot(S, S, ((1,), (1,)))
            SStS = _dot(SSt, S, ((1,), (0,)))
            ca = (1.5 / nrm) * eT_p[i][:, j:j + 1]
            cb = (0.5 / (nrm * nrm * nrm)) * eT_p[i][:, j:j + 1]
            M = aT_p[i][:, j:j + 1] * M - ca * S + cb * SStS
            M_p[i], S_p[i] = M, S
            ys_p[i].append(_dot(kphi_p[i][j:j + 1, :], M, ((1,), (1,))))

    for i in range(pair):
        Y = jnp.concatenate(ys_p[i], axis=0)
        out_ref[i] = _dot(Y, oW_ref[...], ((1,), (1,))) + ob_ref[...]
        M_ref[i] = M_p[i]
        S_ref[i] = S_p[i]


def _run_shard(x, k_aligned, v, M_prev, S_prev, poly_coeffs,
               alpha_W, alpha_b, eta_W, eta_b, theta_W, theta_b,
               gamma_W, gamma_b, out_W, out_b):
    Bl, L, D = x.shape
    nc = L // CHUNK

    row = lambda: pl.BlockSpec((Bl, CHUNK, D), lambda c: (0, c, 0))
    bat = lambda: pl.BlockSpec((Bl, D, D), lambda c: (0, 0, 0))
    fix = lambda s: pl.BlockSpec(s, lambda c: (0,) * len(s))

    return pl.pallas_call(
        _atlas_kernel,
        grid=(nc,),
        in_specs=[
            row(),
            row(),
            row(),
            bat(),
            bat(),
            fix((2, D)),
            fix((D, D)), fix((D, 1)),
            fix((D, D)), fix((D, 1)),
            fix((D, D)), fix((D, 1)),
            fix((D, 1)), fix((1, 1)),
            fix((D, D)), fix((1, D)),
        ],
        out_specs=[
            row(),
            bat(),
            bat(),
        ],
        out_shape=[
            jax.ShapeDtypeStruct((Bl, L, D), jnp.float32),
            jax.ShapeDtypeStruct((Bl, D, D), jnp.float32),
            jax.ShapeDtypeStruct((Bl, D, D), jnp.float32),
        ],
        scratch_shapes=[
            pltpu.VMEM((Bl, W, D), jnp.float32),
            pltpu.VMEM((Bl, W, D), jnp.float32),
        ],
        compiler_params=pltpu.CompilerParams(
            dimension_semantics=("arbitrary",),
        ),
        name="atlas_memory2",
    )(x, k_aligned, v, M_prev, S_prev, poly_coeffs,
      alpha_W, alpha_b, eta_W, eta_b, theta_W, theta_b,
      gamma_W, gamma_b, out_W, out_b)


@jax.jit
def kernel(x, k_aligned, v, M_prev, S_prev, poly_coeffs,
           alpha_W, alpha_b, eta_W, eta_b, theta_W, theta_b,
           gamma_W, gamma_b, out_W, out_b):
    B, L, D = x.shape
    devs = jax.devices()
    nd = 2 if (len(devs) >= 2 and B % 2 == 0) else 1
    mesh = Mesh(np.asarray(devs[:nd]), ("x",))
    sharded = P("x")
    repl = P()
    fn = jax.shard_map(
        _run_shard, mesh=mesh,
        in_specs=(sharded, sharded, sharded, sharded, sharded,
                  repl, repl, repl, repl, repl, repl, repl,
                  repl, repl, repl, repl),
        out_specs=(sharded, sharded, sharded),
        check_vma=False,
    )
    out, M_out, S_out = fn(
        x, k_aligned, v, M_prev, S_prev, poly_coeffs,
        alpha_W, alpha_b.reshape(D, 1), eta_W, eta_b.reshape(D, 1),
        theta_W, theta_b.reshape(D, 1), gamma_W.reshape(D, 1),
        gamma_b.reshape(1, 1), out_W, out_b.reshape(1, D))
    return (out, M_out, S_out)


# speedup vs baseline: 1.4518x; 1.4518x over previous
import jax
import jax.numpy as jnp
from jax.experimental import pallas as pl
from jax.experimental.pallas import tpu as pltpu

W = 8
CHUNK = 8
NS_EPS = 1e-7


def _dot(a, b, dims):
    return jax.lax.dot_general(a, b, (dims, ((), ())),
                               preferred_element_type=jnp.float32)


def _atlas_kernel(x_ref, k_ref, v_ref, Mp_ref, Sp_ref, pc_ref,
                  aW_ref, ab_ref, eW_ref, eb_ref, tW_ref, tb_ref,
                  gW_ref, gb_ref, oW_ref, ob_ref,
                  out_ref, M_ref, S_ref,
                  bk_s, bv_s):
    nb = x_ref.shape[0]
    c = pl.program_id(0)

    @pl.when(c == 0)
    def _init():
        M_ref[...] = Mp_ref[...]
        S_ref[...] = Sp_ref[...]
        bk_s[...] = jnp.zeros_like(bk_s)
        bv_s[...] = jnp.zeros_like(bv_s)

    kphi_p, ks_p, vs_p, a_p, e_p, t_p = [], [], [], [], [], []
    for i in range(nb):
        x_c = x_ref[i]
        k_c = k_ref[i]
        v_c = v_ref[i]
        kphi = pc_ref[0:1, :] * k_c + pc_ref[1:2, :] * (k_c * k_c)
        a_p.append(jax.nn.sigmoid(_dot(x_c, aW_ref[...], ((1,), (1,))) + ab_ref[...]))
        e_p.append(jax.nn.sigmoid(_dot(x_c, eW_ref[...], ((1,), (1,))) + eb_ref[...]) * 0.1)
        t_p.append(jax.nn.sigmoid(_dot(x_c, tW_ref[...], ((1,), (1,))) + tb_ref[...]))
        g = jax.nn.sigmoid(_dot(x_c, gW_ref[...], ((1,), (0,))) + gb_ref[...])
        sg = jnp.sqrt(g)
        kphi_p.append(kphi)
        ks_p.append(sg * kphi)
        vs_p.append(sg * v_c)

    Mt_p = [M_ref[i] for i in range(nb)]
    St_p = [S_ref[i] for i in range(nb)]
    ys_p = [[] for _ in range(nb)]
    for j in range(CHUNK):
        rc = jnp.where(c == 0, jnp.float32(1.0 / (j + 1)), jnp.float32(1.0 / W))
        for i in range(nb):
            bk_s[i, j:j + 1, :] = ks_p[i][j:j + 1, :]
            bv_s[i, j:j + 1, :] = vs_p[i][j:j + 1, :]
            Kw = bk_s[i]
            Vw = bv_s[i]
            Mt, St = Mt_p[i], St_p[i]
            pe = (_dot(Kw, Mt, ((1,), (0,))) - Vw) * rc
            gradT = _dot(Kw, pe, ((0,), (0,)))
            St = t_p[i][j:j + 1, :] * St + gradT
            nrm = jnp.sqrt(jnp.sum(St * St)) + NS_EPS
            SSt = _dot(St, St, ((0,), (0,)))
            TT = _dot(St, SSt, ((1,), (0,)))
            ca = (1.5 / nrm) * e_p[i][j:j + 1, :]
            cb = (0.5 / (nrm * nrm * nrm)) * e_p[i][j:j + 1, :]
            Mt = a_p[i][j:j + 1, :] * Mt - ca * St + cb * TT
            Mt_p[i], St_p[i] = Mt, St
            ys_p[i].append(_dot(kphi_p[i][j:j + 1, :], Mt, ((1,), (0,))))

    for i in range(nb):
        Y = jnp.concatenate(ys_p[i], axis=0)
        out_ref[i] = _dot(Y, oW_ref[...], ((1,), (1,))) + ob_ref[...]
        M_ref[i] = Mt_p[i]
        S_ref[i] = St_p[i]


@jax.jit
def kernel(x, k_aligned, v, M_prev, S_prev, poly_coeffs,
           alpha_W, alpha_b, eta_W, eta_b, theta_W, theta_b,
           gamma_W, gamma_b, out_W, out_b):
    B, L, D = x.shape
    nc = L // CHUNK

    row = lambda: pl.BlockSpec((B, CHUNK, D), lambda c: (0, c, 0))
    bat = lambda: pl.BlockSpec((B, D, D), lambda c: (0, 0, 0))
    fix = lambda s: pl.BlockSpec(s, lambda c: (0,) * len(s))

    out, Mt_out, St_out = pl.pallas_call(
        _atlas_kernel,
        grid=(nc,),
        in_specs=[
            row(),
            row(),
            row(),
            bat(),
            bat(),
            fix((2, D)),
            fix((D, D)), fix((1, D)),
            fix((D, D)), fix((1, D)),
            fix((D, D)), fix((1, D)),
            fix((D, 1)), fix((1, 1)),
            fix((D, D)), fix((1, D)),
        ],
        out_specs=[
            row(),
            bat(),
            bat(),
        ],
        out_shape=[
            jax.ShapeDtypeStruct((B, L, D), jnp.float32),
            jax.ShapeDtypeStruct((B, D, D), jnp.float32),
            jax.ShapeDtypeStruct((B, D, D), jnp.float32),
        ],
        scratch_shapes=[
            pltpu.VMEM((B, W, D), jnp.float32),
            pltpu.VMEM((B, W, D), jnp.float32),
        ],
        compiler_params=pltpu.CompilerParams(
            dimension_semantics=("arbitrary",),
        ),
        name="atlas_memory",
    )(x, k_aligned, v, jnp.swapaxes(M_prev, 1, 2), jnp.swapaxes(S_prev, 1, 2),
      poly_coeffs,
      alpha_W, alpha_b.reshape(1, D), eta_W, eta_b.reshape(1, D),
      theta_W, theta_b.reshape(1, D), gamma_W.reshape(D, 1),
      gamma_b.reshape(1, 1), out_W, out_b.reshape(1, D))
    return (out, jnp.swapaxes(Mt_out, 1, 2), jnp.swapaxes(St_out, 1, 2))


# speedup vs baseline: 2.9937x; 2.0620x over previous
import jax
import jax.numpy as jnp
from jax.experimental import pallas as pl
from jax.experimental.pallas import tpu as pltpu

W = 8
CHUNK = 8
NS_EPS = 1e-7


def _dot(a, b, dims):
    return jax.lax.dot_general(a, b, (dims, ((), ())),
                               preferred_element_type=jnp.float32)


def _atlas_kernel(x_ref, k_ref, v_ref, Mp_ref, Sp_ref, pc_ref,
                  aW_ref, ab_ref, eW_ref, eb_ref, tW_ref, tb_ref,
                  gW_ref, gb_ref, oW_ref, ob_ref,
                  out_ref, M_ref, S_ref,
                  bk_s, bv_s):
    nb = x_ref.shape[0]
    c = pl.program_id(0)

    @pl.when(c == 0)
    def _init():
        M_ref[...] = Mp_ref[...]
        S_ref[...] = Sp_ref[...]
        bk_s[...] = jnp.zeros_like(bk_s)
        bv_s[...] = jnp.zeros_like(bv_s)

    kphi_p, ks_p, vs_p, a_p, e_p, t_p = [], [], [], [], [], []
    for i in range(nb):
        x_c = x_ref[i]
        k_c = k_ref[i]
        v_c = v_ref[i]
        kphi = pc_ref[0:1, :] * k_c + pc_ref[1:2, :] * (k_c * k_c)
        a_p.append(jax.nn.sigmoid(_dot(x_c, aW_ref[...], ((1,), (1,))) + ab_ref[...]))
        e_p.append(jax.nn.sigmoid(_dot(x_c, eW_ref[...], ((1,), (1,))) + eb_ref[...]) * 0.1)
        t_p.append(jax.nn.sigmoid(_dot(x_c, tW_ref[...], ((1,), (1,))) + tb_ref[...]))
        g = jax.nn.sigmoid(_dot(x_c, gW_ref[...], ((1,), (0,))) + gb_ref[...])
        sg = jnp.sqrt(g)
        kphi_p.append(kphi)
        ks_p.append(sg * kphi)
        vs_p.append(sg * v_c)

    Mt_p = [M_ref[i] for i in range(nb)]
    St_p = [S_ref[i] for i in range(nb)]
    ys_p = [[] for _ in range(nb)]
    for j in range(CHUNK):
        rc = jnp.where(c == 0, jnp.float32(1.0 / (j + 1)), jnp.float32(1.0 / W))
        Kw_p, pe_p, SSt_p, TT_p, nrm_p = [None] * nb, [None] * nb, [None] * nb, [None] * nb, [None] * nb
        for i in range(nb):
            bk_s[i, j:j + 1, :] = ks_p[i][j:j + 1, :]
            bv_s[i, j:j + 1, :] = vs_p[i][j:j + 1, :]
            Kw_p[i] = bk_s[i]
        for i in range(nb):
            pe_p[i] = (_dot(Kw_p[i], Mt_p[i], ((1,), (0,))) - bv_s[i]) * rc
        for i in range(nb):
            gradT = _dot(Kw_p[i], pe_p[i], ((0,), (0,)))
            St_p[i] = t_p[i][j:j + 1, :] * St_p[i] + gradT
        for i in range(nb):
            nrm_p[i] = jnp.sqrt(jnp.sum(St_p[i] * St_p[i])) + NS_EPS
            SSt_p[i] = _dot(St_p[i], St_p[i], ((0,), (0,)))
        for i in range(nb):
            TT_p[i] = _dot(St_p[i], SSt_p[i], ((1,), (0,)))
        for i in range(nb):
            nrm = nrm_p[i]
            ca = (1.5 / nrm) * e_p[i][j:j + 1, :]
            cb = (0.5 / (nrm * nrm * nrm)) * e_p[i][j:j + 1, :]
            Mt_p[i] = a_p[i][j:j + 1, :] * Mt_p[i] - ca * St_p[i] + cb * TT_p[i]
        for i in range(nb):
            ys_p[i].append(_dot(kphi_p[i][j:j + 1, :], Mt_p[i], ((1,), (0,))))

    for i in range(nb):
        Y = jnp.concatenate(ys_p[i], axis=0)
        out_ref[i] = _dot(Y, oW_ref[...], ((1,), (1,))) + ob_ref[...]
        M_ref[i] = Mt_p[i]
        S_ref[i] = St_p[i]


@jax.jit
def kernel(x, k_aligned, v, M_prev, S_prev, poly_coeffs,
           alpha_W, alpha_b, eta_W, eta_b, theta_W, theta_b,
           gamma_W, gamma_b, out_W, out_b):
    B, L, D = x.shape
    nc = L // CHUNK

    row = lambda: pl.BlockSpec((B, CHUNK, D), lambda c: (0, c, 0))
    bat = lambda: pl.BlockSpec((B, D, D), lambda c: (0, 0, 0))
    fix = lambda s: pl.BlockSpec(s, lambda c: (0,) * len(s))

    out, Mt_out, St_out = pl.pallas_call(
        _atlas_kernel,
        grid=(nc,),
        in_specs=[
            row(),
            row(),
            row(),
            bat(),
            bat(),
            fix((2, D)),
            fix((D, D)), fix((1, D)),
            fix((D, D)), fix((1, D)),
            fix((D, D)), fix((1, D)),
            fix((D, 1)), fix((1, 1)),
            fix((D, D)), fix((1, D)),
        ],
        out_specs=[
            row(),
            bat(),
            bat(),
        ],
        out_shape=[
            jax.ShapeDtypeStruct((B, L, D), jnp.float32),
            jax.ShapeDtypeStruct((B, D, D), jnp.float32),
            jax.ShapeDtypeStruct((B, D, D), jnp.float32),
        ],
        scratch_shapes=[
            pltpu.VMEM((B, W, D), jnp.float32),
            pltpu.VMEM((B, W, D), jnp.float32),
        ],
        compiler_params=pltpu.CompilerParams(
            dimension_semantics=("arbitrary",),
        ),
        name="atlas_memory",
    )(x, k_aligned, v, jnp.swapaxes(M_prev, 1, 2), jnp.swapaxes(S_prev, 1, 2),
      poly_coeffs,
      alpha_W, alpha_b.reshape(1, D), eta_W, eta_b.reshape(1, D),
      theta_W, theta_b.reshape(1, D), gamma_W.reshape(D, 1),
      gamma_b.reshape(1, 1), out_W, out_b.reshape(1, D))
    return (out, jnp.swapaxes(Mt_out, 1, 2), jnp.swapaxes(St_out, 1, 2))


# speedup vs baseline: 3.1332x; 1.0466x over previous
import jax
import jax.numpy as jnp
from jax.experimental import pallas as pl
from jax.experimental.pallas import tpu as pltpu

W = 8
CHUNK = 8
NS_EPS = 1e-7


def _dot(a, b, dims):
    return jax.lax.dot_general(a, b, (dims, ((), ())),
                               preferred_element_type=jnp.float32)


def _atlas_kernel(x_ref, k_ref, v_ref, Mp_ref, Sp_ref, pc_ref,
                  aW_ref, ab_ref, eW_ref, eb_ref, tW_ref, tb_ref,
                  gW_ref, gb_ref, oW_ref, ob_ref,
                  out_ref, M_ref, S_ref,
                  bk_s, bv_s):
    nb = x_ref.shape[0]
    c = pl.program_id(0)

    @pl.when(c == 0)
    def _init():
        M_ref[...] = Mp_ref[...]
        S_ref[...] = Sp_ref[...]
        bk_s[...] = jnp.zeros_like(bk_s)
        bv_s[...] = jnp.zeros_like(bv_s)

    kphi_p, ks_p, vs_p, a_p, e_p, t_p = [], [], [], [], [], []
    for i in range(nb):
        x_c = x_ref[i]
        k_c = k_ref[i]
        v_c = v_ref[i]
        kphi = pc_ref[0:1, :] * k_c + pc_ref[1:2, :] * (k_c * k_c)
        a_p.append(jax.nn.sigmoid(_dot(x_c, aW_ref[...], ((1,), (1,))) + ab_ref[...]))
        e_p.append(jax.nn.sigmoid(_dot(x_c, eW_ref[...], ((1,), (1,))) + eb_ref[...]) * 0.1)
        t_p.append(jax.nn.sigmoid(_dot(x_c, tW_ref[...], ((1,), (1,))) + tb_ref[...]))
        g = jax.nn.sigmoid(_dot(x_c, gW_ref[...], ((1,), (0,))) + gb_ref[...])
        sg = jnp.sqrt(g)
        kphi_p.append(kphi)
        ks_p.append(sg * kphi)
        vs_p.append(sg * v_c)

    Mt_p = [M_ref[i] for i in range(nb)]
    St_p = [S_ref[i] for i in range(nb)]
    ys_p = [[] for _ in range(nb)]
    for j in range(CHUNK):
        rc = jnp.where(c == 0, jnp.float32(1.0 / (j + 1)), jnp.float32(1.0 / W))
        Kw_p, pe_p, SSt_p, TT_p, nrm_p, Sb_p = ([None] * nb for _ in range(6))
        for i in range(nb):
            bk_s[i, j:j + 1, :] = ks_p[i][j:j + 1, :]
            bv_s[i, j:j + 1, :] = vs_p[i][j:j + 1, :]
            Kw_p[i] = bk_s[i]
        for i in range(nb):
            pe_p[i] = (_dot(Kw_p[i], Mt_p[i], ((1,), (0,))) - bv_s[i]) * rc
        for i in range(nb):
            gradT = _dot(Kw_p[i], pe_p[i], ((0,), (0,)))
            St_p[i] = t_p[i][j:j + 1, :] * St_p[i] + gradT
        for i in range(nb):
            nrm_p[i] = jnp.sqrt(jnp.sum(St_p[i] * St_p[i])) + NS_EPS
            Sb_p[i] = St_p[i].astype(jnp.bfloat16)
            SSt_p[i] = _dot(Sb_p[i], Sb_p[i], ((0,), (0,)))
        for i in range(nb):
            TT_p[i] = _dot(Sb_p[i], SSt_p[i].astype(jnp.bfloat16),
                           ((1,), (0,)))
        for i in range(nb):
            nrm = nrm_p[i]
            ca = (1.5 / nrm) * e_p[i][j:j + 1, :]
            cb = (0.5 / (nrm * nrm * nrm)) * e_p[i][j:j + 1, :]
            Mt_p[i] = a_p[i][j:j + 1, :] * Mt_p[i] - ca * St_p[i] + cb * TT_p[i]
        for i in range(nb):
            ys_p[i].append(_dot(kphi_p[i][j:j + 1, :], Mt_p[i], ((1,), (0,))))

    for i in range(nb):
        Y = jnp.concatenate(ys_p[i], axis=0)
        out_ref[i] = _dot(Y, oW_ref[...], ((1,), (1,))) + ob_ref[...]
        M_ref[i] = Mt_p[i]
        S_ref[i] = St_p[i]


@jax.jit
def kernel(x, k_aligned, v, M_prev, S_prev, poly_coeffs,
           alpha_W, alpha_b, eta_W, eta_b, theta_W, theta_b,
           gamma_W, gamma_b, out_W, out_b):
    B, L, D = x.shape
    nc = L // CHUNK

    row = lambda: pl.BlockSpec((B, CHUNK, D), lambda c: (0, c, 0))
    bat = lambda: pl.BlockSpec((B, D, D), lambda c: (0, 0, 0))
    fix = lambda s: pl.BlockSpec(s, lambda c: (0,) * len(s))

    out, Mt_out, St_out = pl.pallas_call(
        _atlas_kernel,
        grid=(nc,),
        in_specs=[
            row(),
            row(),
            row(),
            bat(),
            bat(),
            fix((2, D)),
            fix((D, D)), fix((1, D)),
            fix((D, D)), fix((1, D)),
            fix((D, D)), fix((1, D)),
            fix((D, 1)), fix((1, 1)),
            fix((D, D)), fix((1, D)),
        ],
        out_specs=[
            row(),
            bat(),
            bat(),
        ],
        out_shape=[
            jax.ShapeDtypeStruct((B, L, D), jnp.float32),
            jax.ShapeDtypeStruct((B, D, D), jnp.float32),
            jax.ShapeDtypeStruct((B, D, D), jnp.float32),
        ],
        scratch_shapes=[
            pltpu.VMEM((B, W, D), jnp.float32),
            pltpu.VMEM((B, W, D), jnp.float32),
        ],
        compiler_params=pltpu.CompilerParams(
            dimension_semantics=("arbitrary",),
        ),
        name="atlas_memory",
    )(x, k_aligned, v, jnp.swapaxes(M_prev, 1, 2), jnp.swapaxes(S_prev, 1, 2),
      poly_coeffs,
      alpha_W, alpha_b.reshape(1, D), eta_W, eta_b.reshape(1, D),
      theta_W, theta_b.reshape(1, D), gamma_W.reshape(D, 1),
      gamma_b.reshape(1, 1), out_W, out_b.reshape(1, D))
    return (out, jnp.swapaxes(Mt_out, 1, 2), jnp.swapaxes(St_out, 1, 2))


# speedup vs baseline: 3.3109x; 1.0567x over previous
import jax
import jax.numpy as jnp
from jax.experimental import pallas as pl
from jax.experimental.pallas import tpu as pltpu

W = 8
CHUNK = 16
NS_EPS = 1e-7


def _dot(a, b, dims):
    return jax.lax.dot_general(a, b, (dims, ((), ())),
                               preferred_element_type=jnp.float32)


def _atlas_kernel(x_ref, k_ref, v_ref, Mp_ref, Sp_ref, pc_ref,
                  aW_ref, ab_ref, eW_ref, eb_ref, tW_ref, tb_ref,
                  gW_ref, gb_ref, oW_ref, ob_ref,
                  out_ref, M_ref, S_ref,
                  bk_s, bv_s):
    nb = x_ref.shape[0]
    c = pl.program_id(0)

    @pl.when(c == 0)
    def _init():
        M_ref[...] = Mp_ref[...]
        S_ref[...] = Sp_ref[...]
        bk_s[...] = jnp.zeros_like(bk_s)
        bv_s[...] = jnp.zeros_like(bv_s)

    kphi_p, ks_p, vs_p, a_p, e_p, t_p, kb_p = [], [], [], [], [], [], []
    for i in range(nb):
        x_c = x_ref[i]
        k_c = k_ref[i]
        v_c = v_ref[i]
        xb = x_c.astype(jnp.bfloat16)
        kphi = pc_ref[0:1, :] * k_c + pc_ref[1:2, :] * (k_c * k_c)
        a_p.append(jax.nn.sigmoid(_dot(xb, aW_ref[...], ((1,), (1,))) + ab_ref[...]))
        e_p.append(jax.nn.sigmoid(_dot(xb, eW_ref[...], ((1,), (1,))) + eb_ref[...]) * 0.1)
        t_p.append(jax.nn.sigmoid(_dot(xb, tW_ref[...], ((1,), (1,))) + tb_ref[...]))
        g = jax.nn.sigmoid(_dot(xb, gW_ref[...], ((1,), (0,))) + gb_ref[...])
        sg = jnp.sqrt(g)
        kphi_p.append(kphi)
        kb_p.append(kphi.astype(jnp.bfloat16))
        ks_p.append((sg * kphi).astype(jnp.bfloat16))
        vs_p.append(sg * v_c)

    Mt_p = [M_ref[i] for i in range(nb)]
    St_p = [S_ref[i] for i in range(nb)]
    Mb_p = [Mt_p[i].astype(jnp.bfloat16) for i in range(nb)]
    ys_p = [[] for _ in range(nb)]
    for j in range(CHUNK):
        jj = j % W
        if j < W:
            rc = jnp.where(c == 0, jnp.float32(1.0 / (j + 1)), jnp.float32(1.0 / W))
        else:
            rc = jnp.float32(1.0 / W)
        Kw_p, pe_p, SSt_p, TT_p, nrm_p, Sb_p = ([None] * nb for _ in range(6))
        for i in range(nb):
            bk_s[i, jj:jj + 1, :] = ks_p[i][j:j + 1, :]
            bv_s[i, jj:jj + 1, :] = vs_p[i][j:j + 1, :]
            Kw_p[i] = bk_s[i]
        for i in range(nb):
            pe_p[i] = (_dot(Kw_p[i], Mb_p[i], ((1,), (0,))) - bv_s[i]) * rc
        for i in range(nb):
            gradT = _dot(Kw_p[i], pe_p[i].astype(jnp.bfloat16),
                         ((0,), (0,)))
            St_p[i] = t_p[i][j:j + 1, :] * St_p[i] + gradT
        for i in range(nb):
            nrm_p[i] = jnp.sqrt(jnp.sum(St_p[i] * St_p[i])) + NS_EPS
            Sb_p[i] = St_p[i].astype(jnp.bfloat16)
            SSt_p[i] = _dot(Sb_p[i], Sb_p[i], ((0,), (0,)))
        for i in range(nb):
            TT_p[i] = _dot(Sb_p[i], SSt_p[i].astype(jnp.bfloat16),
                           ((1,), (0,)))
        for i in range(nb):
            nrm = nrm_p[i]
            ca = (1.5 / nrm) * e_p[i][j:j + 1, :]
            cb = (0.5 / (nrm * nrm * nrm)) * e_p[i][j:j + 1, :]
            Mt_p[i] = a_p[i][j:j + 1, :] * Mt_p[i] - ca * St_p[i] + cb * TT_p[i]
            Mb_p[i] = Mt_p[i].astype(jnp.bfloat16)
        for i in range(nb):
            ys_p[i].append(_dot(kb_p[i][j:j + 1, :], Mb_p[i], ((1,), (0,))))

    for i in range(nb):
        Y = jnp.concatenate(ys_p[i], axis=0)
        out_ref[i] = _dot(Y.astype(jnp.bfloat16), oW_ref[...],
                          ((1,), (1,))) + ob_ref[...]
        M_ref[i] = Mt_p[i]
        S_ref[i] = St_p[i]


@jax.jit
def kernel(x, k_aligned, v, M_prev, S_prev, poly_coeffs,
           alpha_W, alpha_b, eta_W, eta_b, theta_W, theta_b,
           gamma_W, gamma_b, out_W, out_b):
    B, L, D = x.shape
    nc = L // CHUNK

    row = lambda: pl.BlockSpec((B, CHUNK, D), lambda c: (0, c, 0))
    bat = lambda: pl.BlockSpec((B, D, D), lambda c: (0, 0, 0))
    fix = lambda s: pl.BlockSpec(s, lambda c: (0,) * len(s))

    out, Mt_out, St_out = pl.pallas_call(
        _atlas_kernel,
        grid=(nc,),
        in_specs=[
            row(),
            row(),
            row(),
            bat(),
            bat(),
            fix((2, D)),
            fix((D, D)), fix((1, D)),
            fix((D, D)), fix((1, D)),
            fix((D, D)), fix((1, D)),
            fix((D, 1)), fix((1, 1)),
            fix((D, D)), fix((1, D)),
        ],
        out_specs=[
            row(),
            bat(),
            bat(),
        ],
        out_shape=[
            jax.ShapeDtypeStruct((B, L, D), jnp.float32),
            jax.ShapeDtypeStruct((B, D, D), jnp.float32),
            jax.ShapeDtypeStruct((B, D, D), jnp.float32),
        ],
        scratch_shapes=[
            pltpu.VMEM((B, W, D), jnp.bfloat16),
            pltpu.VMEM((B, W, D), jnp.float32),
        ],
        compiler_params=pltpu.CompilerParams(
            dimension_semantics=("arbitrary",),
        ),
        name="atlas_memory",
    )(x, k_aligned, v, jnp.swapaxes(M_prev, 1, 2), jnp.swapaxes(S_prev, 1, 2),
      poly_coeffs,
      alpha_W.astype(jnp.bfloat16), alpha_b.reshape(1, D),
      eta_W.astype(jnp.bfloat16), eta_b.reshape(1, D),
      theta_W.astype(jnp.bfloat16), theta_b.reshape(1, D),
      gamma_W.reshape(D, 1).astype(jnp.bfloat16),
      gamma_b.reshape(1, 1), out_W.astype(jnp.bfloat16), out_b.reshape(1, D))
    return (out, jnp.swapaxes(Mt_out, 1, 2), jnp.swapaxes(St_out, 1, 2))


# speedup vs baseline: 3.6615x; 1.1059x over previous
import jax
import jax.numpy as jnp
from jax.experimental import pallas as pl
from jax.experimental.pallas import tpu as pltpu

W = 8
CHUNK = 16
NS_EPS = 1e-7


def _dot(a, b, dims):
    return jax.lax.dot_general(a, b, (dims, ((), ())),
                               preferred_element_type=jnp.float32)


def _atlas_kernel(x_ref, k_ref, v_ref, Mp_ref, Sp_ref, pc_ref,
                  aW_ref, ab_ref, eW_ref, eb_ref, tW_ref, tb_ref,
                  gW_ref, gb_ref, oW_ref, ob_ref,
                  out_ref, M_ref, S_ref,
                  bk_s, bv_s):
    nb = x_ref.shape[0]
    c = pl.program_id(0)

    @pl.when(c == 0)
    def _init():
        M_ref[...] = Mp_ref[...]
        S_ref[...] = Sp_ref[...]
        bk_s[...] = jnp.zeros_like(bk_s)
        bv_s[...] = jnp.zeros_like(bv_s)

    kphi_p, ks_p, vs_p, a_p, e_p, t_p, kb_p = [], [], [], [], [], [], []
    for i in range(nb):
        x_c = x_ref[i]
        k_c = k_ref[i]
        v_c = v_ref[i]
        xb = x_c.astype(jnp.bfloat16)
        kphi = pc_ref[0:1, :] * k_c + pc_ref[1:2, :] * (k_c * k_c)
        a_p.append(jax.nn.sigmoid(_dot(xb, aW_ref[...], ((1,), (1,))) + ab_ref[...]))
        e_p.append(jax.nn.sigmoid(_dot(xb, eW_ref[...], ((1,), (1,))) + eb_ref[...]) * 0.1)
        t_p.append(jax.nn.sigmoid(_dot(xb, tW_ref[...], ((1,), (1,))) + tb_ref[...]))
        g = jax.nn.sigmoid(_dot(xb, gW_ref[...], ((1,), (0,))) + gb_ref[...])
        sg = jnp.sqrt(g)
        kphi_p.append(kphi)
        kb_p.append(kphi.astype(jnp.bfloat16))
        ks_p.append((sg * kphi).astype(jnp.bfloat16))
        vs_p.append(sg * v_c)

    Mt_p = [M_ref[i] for i in range(nb)]
    St_p = [S_ref[i] for i in range(nb)]
    Mb_p = [Mt_p[i].astype(jnp.bfloat16) for i in range(nb)]
    ys_p = [[] for _ in range(nb)]
    for j in range(CHUNK):
        jj = j % W
        if j < W:
            rc = jnp.where(c == 0, jnp.float32(1.0 / (j + 1)), jnp.float32(1.0 / W))
        else:
            rc = jnp.float32(1.0 / W)
        Kw_p, pe_p, SSt_p, TT_p, nrm_p, Sb_p = ([None] * nb for _ in range(6))
        for i in range(nb):
            bk_s[i, jj:jj + 1, :] = ks_p[i][j:j + 1, :]
            bv_s[i, jj:jj + 1, :] = vs_p[i][j:j + 1, :]
            Kw_p[i] = bk_s[i]
        for i in range(nb):
            if j > 0:
                ys_p[i].append(_dot(kb_p[i][j - 1:j, :], Mb_p[i], ((1,), (0,))))
            pe_p[i] = (_dot(Kw_p[i], Mb_p[i], ((1,), (0,))) - bv_s[i]) * rc
        for i in range(nb):
            gradT = _dot(Kw_p[i], pe_p[i].astype(jnp.bfloat16),
                         ((0,), (0,)))
            St_p[i] = t_p[i][j:j + 1, :] * St_p[i] + gradT
        for i in range(nb):
            nrm_p[i] = jnp.sqrt(jnp.sum(St_p[i] * St_p[i])) + NS_EPS
            Sb_p[i] = St_p[i].astype(jnp.bfloat16)
            SSt_p[i] = _dot(Sb_p[i], Sb_p[i], ((0,), (0,)))
        for i in range(nb):
            TT_p[i] = _dot(Sb_p[i], SSt_p[i].astype(jnp.bfloat16),
                           ((1,), (0,)))
        for i in range(nb):
            nrm = nrm_p[i]
            ca = (1.5 / nrm) * e_p[i][j:j + 1, :]
            cb = (0.5 / (nrm * nrm * nrm)) * e_p[i][j:j + 1, :]
            Mt_p[i] = a_p[i][j:j + 1, :] * Mt_p[i] - ca * St_p[i] + cb * TT_p[i]
            Mb_p[i] = Mt_p[i].astype(jnp.bfloat16)

    for i in range(nb):
        ys_p[i].append(_dot(kb_p[i][CHUNK - 1:CHUNK, :], Mb_p[i], ((1,), (0,))))
        Y = jnp.concatenate(ys_p[i], axis=0)
        out_ref[i] = _dot(Y.astype(jnp.bfloat16), oW_ref[...],
                          ((1,), (1,))) + ob_ref[...]
        M_ref[i] = Mt_p[i]
        S_ref[i] = St_p[i]


@jax.jit
def kernel(x, k_aligned, v, M_prev, S_prev, poly_coeffs,
           alpha_W, alpha_b, eta_W, eta_b, theta_W, theta_b,
           gamma_W, gamma_b, out_W, out_b):
    B, L, D = x.shape
    nc = L // CHUNK

    row = lambda: pl.BlockSpec((B, CHUNK, D), lambda c: (0, c, 0))
    bat = lambda: pl.BlockSpec((B, D, D), lambda c: (0, 0, 0))
    fix = lambda s: pl.BlockSpec(s, lambda c: (0,) * len(s))

    out, Mt_out, St_out = pl.pallas_call(
        _atlas_kernel,
        grid=(nc,),
        in_specs=[
            row(),
            row(),
            row(),
            bat(),
            bat(),
            fix((2, D)),
            fix((D, D)), fix((1, D)),
            fix((D, D)), fix((1, D)),
            fix((D, D)), fix((1, D)),
            fix((D, 1)), fix((1, 1)),
            fix((D, D)), fix((1, D)),
        ],
        out_specs=[
            row(),
            bat(),
            bat(),
        ],
        out_shape=[
            jax.ShapeDtypeStruct((B, L, D), jnp.float32),
            jax.ShapeDtypeStruct((B, D, D), jnp.float32),
            jax.ShapeDtypeStruct((B, D, D), jnp.float32),
        ],
        scratch_shapes=[
            pltpu.VMEM((B, W, D), jnp.bfloat16),
            pltpu.VMEM((B, W, D), jnp.float32),
        ],
        compiler_params=pltpu.CompilerParams(
            dimension_semantics=("arbitrary",),
        ),
        name="atlas_memory",
    )(x, k_aligned, v, jnp.swapaxes(M_prev, 1, 2), jnp.swapaxes(S_prev, 1, 2),
      poly_coeffs,
      alpha_W.astype(jnp.bfloat16), alpha_b.reshape(1, D),
      eta_W.astype(jnp.bfloat16), eta_b.reshape(1, D),
      theta_W.astype(jnp.bfloat16), theta_b.reshape(1, D),
      gamma_W.reshape(D, 1).astype(jnp.bfloat16),
      gamma_b.reshape(1, 1), out_W.astype(jnp.bfloat16), out_b.reshape(1, D))
    return (out, jnp.swapaxes(Mt_out, 1, 2), jnp.swapaxes(St_out, 1, 2))


# speedup vs baseline: 4.1647x; 1.1374x over previous
import jax
import jax.numpy as jnp
from jax.experimental import pallas as pl
from jax.experimental.pallas import tpu as pltpu

W = 8
CHUNK = 16
NS_EPS = 1e-7


def _dot(a, b, dims):
    return jax.lax.dot_general(a, b, (dims, ((), ())),
                               preferred_element_type=jnp.float32)


def _atlas_kernel(x_ref, k_ref, v_ref, Mp_ref, Sp_ref, pc_ref,
                  aW_ref, ab_ref, eW_ref, eb_ref, tW_ref, tb_ref,
                  gW_ref, gb_ref, oW_ref, ob_ref,
                  out_ref, M_ref, S_ref,
                  bk_s, bv_s):
    nb = x_ref.shape[0]
    c = pl.program_id(0)

    @pl.when(c == 0)
    def _init():
        M_ref[...] = Mp_ref[...]
        S_ref[...] = Sp_ref[...]
        bk_s[...] = jnp.zeros_like(bk_s)
        bv_s[...] = jnp.zeros_like(bv_s)

    aW = aW_ref[...]
    eW = eW_ref[...]
    tW = tW_ref[...]
    gW = gW_ref[...]
    kphi_p, ks_p, vs_p, a_p, e_p, t_p, kb_p = [], [], [], [], [], [], []
    for i in range(nb):
        x_c = x_ref[i]
        k_c = k_ref[i]
        v_c = v_ref[i]
        xb = x_c.astype(jnp.bfloat16)
        kphi = pc_ref[0:1, :] * k_c + pc_ref[1:2, :] * (k_c * k_c)
        a_p.append(jax.nn.sigmoid(_dot(xb, aW, ((1,), (1,))) + ab_ref[...]))
        e_p.append(jax.nn.sigmoid(_dot(xb, eW, ((1,), (1,))) + eb_ref[...]) * 0.1)
        t_p.append(jax.nn.sigmoid(_dot(xb, tW, ((1,), (1,))) + tb_ref[...]))
        g = jax.nn.sigmoid(_dot(xb, gW, ((1,), (0,))) + gb_ref[...])
        sg = jnp.sqrt(g)
        kphi_p.append(kphi)
        kb_p.append(kphi.astype(jnp.bfloat16))
        ks_p.append((sg * kphi).astype(jnp.bfloat16))
        vs_p.append(sg * v_c)

    Mt_p = [M_ref[i] for i in range(nb)]
    St_p = [S_ref[i] for i in range(nb)]
    Mb_p = [Mt_p[i].astype(jnp.bfloat16) for i in range(nb)]
    ys_p = [[] for _ in range(nb)]
    for j in range(CHUNK):
        jj = j % W
        if j < W:
            rc = jnp.where(c == 0, jnp.float32(1.0 / (j + 1)), jnp.float32(1.0 / W))
        else:
            rc = jnp.float32(1.0 / W)
        Kw_p, pe_p, SSt_p, TT_p, nrm_p, Sb_p = ([None] * nb for _ in range(6))
        for i in range(nb):
            bk_s[i, jj:jj + 1, :] = ks_p[i][j:j + 1, :]
            bv_s[i, jj:jj + 1, :] = vs_p[i][j:j + 1, :]
            Kw_p[i] = bk_s[i]
        for i in range(nb):
            if j > 0:
                ys_p[i].append(_dot(kb_p[i][j - 1:j, :], Mb_p[i], ((1,), (0,))))
            pe_p[i] = (_dot(Kw_p[i], Mb_p[i], ((1,), (0,))) - bv_s[i]) * rc
        for i in range(nb):
            gradT = _dot(Kw_p[i], pe_p[i].astype(jnp.bfloat16),
                         ((0,), (0,)))
            St_p[i] = t_p[i][j:j + 1, :] * St_p[i] + gradT
        for i in range(nb):
            nrm_p[i] = jnp.sqrt(jnp.sum(St_p[i] * St_p[i])) + NS_EPS
            Sb_p[i] = St_p[i].astype(jnp.bfloat16)
            SSt_p[i] = _dot(Sb_p[i], Sb_p[i], ((1,), (1,)))
        for i in range(nb):
            TT_p[i] = _dot(SSt_p[i].astype(jnp.bfloat16), Sb_p[i],
                           ((1,), (0,)))
        for i in range(nb):
            nrm = nrm_p[i]
            ca = (1.5 / nrm) * e_p[i][j:j + 1, :]
            cb = (0.5 / (nrm * nrm * nrm)) * e_p[i][j:j + 1, :]
            Mt_p[i] = a_p[i][j:j + 1, :] * Mt_p[i] - ca * St_p[i] + cb * TT_p[i]
            Mb_p[i] = Mt_p[i].astype(jnp.bfloat16)

    oW = oW_ref[...]
    ob = ob_ref[...]
    for i in range(nb):
        ys_p[i].append(_dot(kb_p[i][CHUNK - 1:CHUNK, :], Mb_p[i], ((1,), (0,))))
        Y = jnp.concatenate(ys_p[i], axis=0)
        out_ref[i] = _dot(Y.astype(jnp.bfloat16), oW, ((1,), (1,))) + ob
        M_ref[i] = Mt_p[i]
        S_ref[i] = St_p[i]


@jax.jit
def kernel(x, k_aligned, v, M_prev, S_prev, poly_coeffs,
           alpha_W, alpha_b, eta_W, eta_b, theta_W, theta_b,
           gamma_W, gamma_b, out_W, out_b):
    B, L, D = x.shape
    nc = L // CHUNK

    row = lambda: pl.BlockSpec((B, CHUNK, D), lambda c: (0, c, 0))
    bat = lambda: pl.BlockSpec((B, D, D), lambda c: (0, 0, 0))
    fix = lambda s: pl.BlockSpec(s, lambda c: (0,) * len(s))

    out, Mt_out, St_out = pl.pallas_call(
        _atlas_kernel,
        grid=(nc,),
        in_specs=[
            row(),
            row(),
            row(),
            bat(),
            bat(),
            fix((2, D)),
            fix((D, D)), fix((1, D)),
            fix((D, D)), fix((1, D)),
            fix((D, D)), fix((1, D)),
            fix((D, 1)), fix((1, 1)),
            fix((D, D)), fix((1, D)),
        ],
        out_specs=[
            row(),
            bat(),
            bat(),
        ],
        out_shape=[
            jax.ShapeDtypeStruct((B, L, D), jnp.float32),
            jax.ShapeDtypeStruct((B, D, D), jnp.float32),
            jax.ShapeDtypeStruct((B, D, D), jnp.float32),
        ],
        scratch_shapes=[
            pltpu.VMEM((B, W, D), jnp.bfloat16),
            pltpu.VMEM((B, W, D), jnp.float32),
        ],
        compiler_params=pltpu.CompilerParams(
            dimension_semantics=("arbitrary",),
        ),
        name="atlas_memory",
    )(x, k_aligned, v, jnp.swapaxes(M_prev, 1, 2), jnp.swapaxes(S_prev, 1, 2),
      poly_coeffs,
      alpha_W.astype(jnp.bfloat16), alpha_b.reshape(1, D),
      eta_W.astype(jnp.bfloat16), eta_b.reshape(1, D),
      theta_W.astype(jnp.bfloat16), theta_b.reshape(1, D),
      gamma_W.reshape(D, 1).astype(jnp.bfloat16),
      gamma_b.reshape(1, 1), out_W.astype(jnp.bfloat16), out_b.reshape(1, D))
    return (out, jnp.swapaxes(Mt_out, 1, 2), jnp.swapaxes(St_out, 1, 2))
